# baseline (speedup 1.0000x reference)
import functools

import jax
import jax.numpy as jnp
from jax import lax
from jax.experimental import pallas as pl
from jax.experimental.pallas import tpu as pltpu
from jax.experimental.pallas import tpu_sc as plsc

BATCH = 16384
EMB = 32
HID = 64
IDX_CHUNK = 128


def _sc_gather(users, items, user_table, item_table):
    mesh = plsc.VectorSubcoreMesh(core_axis_name="c", subcore_axis_name="s")
    nw = mesh.num_cores * mesh.num_subcores
    b_per_w = BATCH // nw
    n_chunks = b_per_w // IDX_CHUNK

    @functools.partial(
        pl.kernel,
        out_type=(
            jax.ShapeDtypeStruct((BATCH, EMB), jnp.float32),
            jax.ShapeDtypeStruct((BATCH, EMB), jnp.float32),
        ),
        mesh=mesh,
        scratch_types=[
            pltpu.VMEM((n_chunks, IDX_CHUNK), jnp.int32),
            pltpu.VMEM((n_chunks, IDX_CHUNK), jnp.int32),
            pltpu.VMEM((b_per_w, EMB), jnp.float32),
            pltpu.VMEM((b_per_w, EMB), jnp.float32),
            pltpu.SemaphoreType.DMA,
        ],
        compiler_params=pltpu.CompilerParams(use_tc_tiling_on_sc=False),
    )
    def gather_kernel(users_hbm, items_hbm, ut_hbm, it_hbm, ue_hbm, ie_hbm,
                      uidx_v, iidx_v, urows_v, irows_v, sem):
        wid = lax.axis_index("s") * mesh.num_cores + lax.axis_index("c")
        base = wid * b_per_w
        for j in range(n_chunks):
            off = base + j * IDX_CHUNK
            pltpu.sync_copy(users_hbm.at[pl.ds(off, IDX_CHUNK)], uidx_v.at[j])
            pltpu.sync_copy(items_hbm.at[pl.ds(off, IDX_CHUNK)], iidx_v.at[j])
        copies = []
        for j in range(n_chunks):
            dst = pl.ds(j * IDX_CHUNK, IDX_CHUNK)
            copies.append(pltpu.async_copy(ut_hbm.at[uidx_v.at[j]], urows_v.at[dst], sem))
            copies.append(pltpu.async_copy(it_hbm.at[iidx_v.at[j]], irows_v.at[dst], sem))
        for c in copies:
            c.wait()
        pltpu.sync_copy(urows_v, ue_hbm.at[pl.ds(base, b_per_w)])
        pltpu.sync_copy(irows_v, ie_hbm.at[pl.ds(base, b_per_w)])

    return gather_kernel(users, items, user_table, item_table)


def _mlp_kernel(ue_ref, ie_ref, w1u_ref, w1i_ref, b1_ref, w2_ref, b2_ref, out_ref):
    h = (jnp.dot(ue_ref[...], w1u_ref[...], preferred_element_type=jnp.float32)
         + jnp.dot(ie_ref[...], w1i_ref[...], preferred_element_type=jnp.float32)
         + b1_ref[...])
    h = jnp.maximum(h, 0.0)
    out_ref[...] = jnp.dot(h, w2_ref[...], preferred_element_type=jnp.float32) + b2_ref[...]


def _tc_mlp(ue, ie, W1, b1, W2, b2):
    rows = 2048
    grid = BATCH // rows
    w1u = W1[:, :EMB].T
    w1i = W1[:, EMB:].T
    out = pl.pallas_call(
        _mlp_kernel,
        grid=(grid,),
        in_specs=[
            pl.BlockSpec((rows, EMB), lambda i: (i, 0)),
            pl.BlockSpec((rows, EMB), lambda i: (i, 0)),
            pl.BlockSpec((EMB, HID), lambda i: (0, 0)),
            pl.BlockSpec((EMB, HID), lambda i: (0, 0)),
            pl.BlockSpec((1, HID), lambda i: (0, 0)),
            pl.BlockSpec((HID, 1), lambda i: (0, 0)),
            pl.BlockSpec((1, 1), lambda i: (0, 0)),
        ],
        out_specs=pl.BlockSpec((rows, 1), lambda i: (i, 0)),
        out_shape=jax.ShapeDtypeStruct((BATCH, 1), jnp.float32),
    )(ue, ie, w1u, w1i, b1.reshape(1, HID), W2.T, b2.reshape(1, 1))
    return out[:, 0]


def kernel(users, items, user_table, item_table, W1, b1, W2, b2):
    ue, ie = _sc_gather(users, items, user_table, item_table)
    return _tc_mlp(ue, ie, W1, b1, W2, b2)


# speedup vs baseline: 3.3507x; 3.3507x over previous
import functools

import jax
import jax.numpy as jnp
from jax import lax
from jax.experimental import pallas as pl
from jax.experimental.pallas import tpu as pltpu
from jax.experimental.pallas import tpu_sc as plsc

BATCH = 16384
EMB = 32
HID = 64
GRP = 16


def _sc_gather(users, items, user_tableT, item_tableT):
    mesh = plsc.VectorSubcoreMesh(core_axis_name="c", subcore_axis_name="s")
    nw = mesh.num_cores * mesh.num_subcores
    b_per_w = BATCH // nw

    @functools.partial(
        pl.kernel,
        out_type=(
            jax.ShapeDtypeStruct((EMB, BATCH), jnp.float32),
            jax.ShapeDtypeStruct((EMB, BATCH), jnp.float32),
        ),
        mesh=mesh,
        scratch_types=[
            pltpu.VMEM((b_per_w,), jnp.int32),
            pltpu.VMEM((b_per_w,), jnp.int32),
            pltpu.VMEM((GRP, EMB, 128), jnp.float32),
            pltpu.VMEM((EMB, b_per_w), jnp.float32),
            pltpu.VMEM((EMB, b_per_w), jnp.float32),
            pltpu.SemaphoreType.DMA,
        ],
        compiler_params=pltpu.CompilerParams(
            use_tc_tiling_on_sc=True, needs_layout_passes=False),
    )
    def gather_kernel(users_hbm, items_hbm, ut_hbm, it_hbm, ue_hbm, ie_hbm,
                      uidx_v, iidx_v, blks_v, ucols_v, icols_v, sem):
        wid = lax.axis_index("s") * mesh.num_cores + lax.axis_index("c")
        base = pl.multiple_of(wid * b_per_w, b_per_w)
        pltpu.sync_copy(users_hbm.at[pl.ds(base, b_per_w)], uidx_v)
        pltpu.sync_copy(items_hbm.at[pl.ds(base, b_per_w)], iidx_v)

        lane = lax.iota(jnp.int32, 16)

        def process(idx_v, tbl_hbm, cols_v):
            def group(g, _):
                off = pl.multiple_of(g * GRP, GRP)
                v = idx_v[pl.ds(off, GRP)]
                cols = []
                for j in range(GRP):
                    c = jnp.sum(jnp.where(lane == j, v, 0))
                    cols.append(c)
                    blk = pl.multiple_of((c // 128) * 128, 128)
                    pltpu.async_copy(
                        tbl_hbm.at[:, pl.ds(blk, 128)], blks_v.at[j], sem)
                for j in range(GRP):
                    pltpu.make_async_copy(
                        tbl_hbm.at[:, pl.ds(0, 128)], blks_v.at[j], sem
                    ).wait()
                for j in range(GRP):
                    lv = jnp.full((16,), cols[j] % 128, jnp.int32)
                    top = plsc.load_gather(blks_v.at[j], [lane, lv])
                    bot = plsc.load_gather(blks_v.at[j], [lane + 16, lv])
                    dst = jnp.full((16,), off + j, jnp.int32)
                    plsc.store_scatter(cols_v, [lane, dst], top)
                    plsc.store_scatter(cols_v, [lane + 16, dst], bot)
                return 0
            lax.fori_loop(0, b_per_w // GRP, group, 0)

        process(uidx_v, ut_hbm, ucols_v)
        process(iidx_v, it_hbm, icols_v)

        pltpu.sync_copy(ucols_v, ue_hbm.at[:, pl.ds(base, b_per_w)])
        pltpu.sync_copy(icols_v, ie_hbm.at[:, pl.ds(base, b_per_w)])

    return gather_kernel(users, items, user_tableT, item_tableT)


def _mlp_kernel(ueT_ref, ieT_ref, w1u_ref, w1i_ref, b1_ref, w2_ref, b2_ref, out_ref):
    h = (jnp.dot(w1u_ref[...], ueT_ref[...], preferred_element_type=jnp.float32)
         + jnp.dot(w1i_ref[...], ieT_ref[...], preferred_element_type=jnp.float32)
         + b1_ref[...])
    h = jnp.maximum(h, 0.0)
    out_ref[...] = jnp.dot(w2_ref[...], h, preferred_element_type=jnp.float32) + b2_ref[...]


def _tc_mlp(ueT, ieT, W1, b1, W2, b2):
    cols = 2048
    grid = BATCH // cols
    w1u = W1[:, :EMB]
    w1i = W1[:, EMB:]
    out = pl.pallas_call(
        _mlp_kernel,
        grid=(grid,),
        in_specs=[
            pl.BlockSpec((EMB, cols), lambda i: (0, i)),
            pl.BlockSpec((EMB, cols), lambda i: (0, i)),
            pl.BlockSpec((HID, EMB), lambda i: (0, 0)),
            pl.BlockSpec((HID, EMB), lambda i: (0, 0)),
            pl.BlockSpec((HID, 1), lambda i: (0, 0)),
            pl.BlockSpec((1, HID), lambda i: (0, 0)),
            pl.BlockSpec((1, 1), lambda i: (0, 0)),
        ],
        out_specs=pl.BlockSpec((1, cols), lambda i: (0, i)),
        out_shape=jax.ShapeDtypeStruct((1, BATCH), jnp.float32),
    )(ueT, ieT, w1u, w1i, b1.reshape(HID, 1), W2, b2.reshape(1, 1))
    return out[0]


def kernel(users, items, user_table, item_table, W1, b1, W2, b2):
    ueT, ieT = _sc_gather(users, items, user_table.T, item_table.T)
    return _tc_mlp(ueT, ieT, W1, b1, W2, b2)
